# Initial kernel scaffold; baseline (speedup 1.0000x reference)
import jax
import jax.numpy as jnp
from jax.experimental import pallas as pl


def kernel(features, indices, grid_coord, offset):
    raise NotImplementedError("write your pallas kernel here")


# speedup vs baseline: 1.4322x; 1.4322x over previous
import functools

import jax
import jax.numpy as jnp
from jax import lax
from jax.experimental import pallas as pl
from jax.experimental.pallas import tpu as pltpu
from jax.experimental.pallas import tpu_sc as plsc

C = 32
NPTS = 200000
NNZ = 100000
DGRID = 64
DHW = DGRID * DGRID * DGRID
TBL = 2 * DHW

NC, NS, L = 2, 16, 16
NW = NC * NS

PTS_P = 200704
PTS_W = PTS_P // NW
N_CHUNK = 4
CHUNK = PTS_W // N_CHUNK

NNZ_P = 100352
NNZ_W = NNZ_P // NS
NPAD_ROWS = NNZ_P - NNZ
TBL_P = TBL + NPAD_ROWS

SENTINEL = NNZ
FILLBUF = 4096
FILL_W = TBL // NS


def _body(feats_hbm, ind_hbm, gc_hbm, off_hbm, out_hbm,
          table_sh, fillbuf, colbuf, vidx, vals, tvals, rows, offv_v, sem):
    core = lax.axis_index("c")
    sub = lax.axis_index("s")
    wid = sub * NC + core
    iota = lax.iota(jnp.int32, L)

    def fill_vec(j, _):
        fillbuf[pl.ds(j * L, L)] = jnp.full((L,), SENTINEL, jnp.int32)
        return _
    lax.fori_loop(0, FILLBUF // L, fill_vec, 0)
    for r in range(FILL_W // FILLBUF):
        pltpu.sync_copy(fillbuf, table_sh.at[pl.ds(sub * FILL_W + r * FILLBUF,
                                                   FILLBUF)])

    rbase = sub * NNZ_W
    for k in range(4):
        pltpu.sync_copy(ind_hbm.at[pl.ds(k * NNZ_P + rbase, NNZ_W)], colbuf)

        def horner(j, _, first=(k == 0)):
            c16 = colbuf[pl.ds(j * L, L)]
            if first:
                vidx[pl.ds(j * L, L)] = c16
            else:
                vidx[pl.ds(j * L, L)] = vidx[pl.ds(j * L, L)] * DGRID + c16
            return _
        lax.fori_loop(0, NNZ_W // L, horner, 0)

    def mkvals(j, _):
        vals[pl.ds(j * L, L)] = iota + (rbase + j * L)
        return _
    lax.fori_loop(0, NNZ_W // L, mkvals, 0)

    plsc.subcore_barrier()
    pltpu.sync_copy(vals, table_sh.at[vidx])
    plsc.subcore_barrier()

    pltpu.sync_copy(off_hbm, offv_v)
    offv = offv_v[...]
    pbase = wid * PTS_W

    def init_batch(j, _):
        pid = iota + (pbase + j * L)
        vidx[pl.ds(j * L, L)] = jnp.where(pid >= offv, 1, 0).astype(jnp.int32)
        return _
    lax.fori_loop(0, PTS_W // L, init_batch, 0)

    for k in range(3):
        pltpu.sync_copy(gc_hbm.at[pl.ds(k * PTS_P + pbase, PTS_W)], colbuf)

        def hornerq(j, _):
            vidx[pl.ds(j * L, L)] = (vidx[pl.ds(j * L, L)] * DGRID
                                     + colbuf[pl.ds(j * L, L)])
            return _
        lax.fori_loop(0, PTS_W // L, hornerq, 0)

    pltpu.sync_copy(table_sh.at[vidx], tvals)

    for cchunk in range(N_CHUNK):
        coff = cchunk * CHUNK
        pltpu.async_copy(feats_hbm.at[tvals.at[pl.ds(coff, CHUNK)]],
                         rows, sem).wait()
        pltpu.sync_copy(rows, out_hbm.at[pl.ds(pbase + coff, CHUNK)])


@functools.partial(jax.jit, static_argnames=())
def kernel(features, indices, grid_coord, offset):
    feats_ext = jnp.concatenate(
        [features, jnp.zeros((1, C), features.dtype)], axis=0)

    j = jnp.arange(NPAD_ROWS, dtype=jnp.int32)
    pad = jnp.stack([jnp.full_like(j, 2), jnp.zeros_like(j),
                     jnp.zeros_like(j), j], axis=1)
    ind_pad = jnp.concatenate([indices.astype(jnp.int32), pad], axis=0)
    ind_t = ind_pad.T.reshape(-1)

    gc_pad = jnp.concatenate(
        [grid_coord.astype(jnp.int32),
         jnp.zeros((PTS_P - NPTS, 3), jnp.int32)], axis=0)
    gc_t = gc_pad.T.reshape(-1)

    off_vec = jnp.broadcast_to(offset[0].astype(jnp.int32), (L,))

    mesh = plsc.VectorSubcoreMesh(core_axis_name="c", subcore_axis_name="s",
                                  num_cores=NC, num_subcores=NS)
    out = pl.kernel(
        _body,
        out_type=jax.ShapeDtypeStruct((PTS_P, C), jnp.float32),
        mesh=mesh,
        compiler_params=pltpu.CompilerParams(use_tc_tiling_on_sc=False),
        scratch_types=[
            pltpu.VMEM_SHARED((TBL_P,), jnp.int32),
            pltpu.VMEM((FILLBUF,), jnp.int32),
            pltpu.VMEM((PTS_W,), jnp.int32),
            pltpu.VMEM((PTS_W,), jnp.int32),
            pltpu.VMEM((NNZ_W,), jnp.int32),
            pltpu.VMEM((PTS_W,), jnp.int32),
            pltpu.VMEM((CHUNK, C), jnp.float32),
            pltpu.VMEM((L,), jnp.int32),
            pltpu.SemaphoreType.DMA,
        ],
    )(feats_ext, ind_t, gc_t, off_vec)
    return lax.stop_gradient(out[:NPTS])
